# baseline (speedup 1.0000x reference)
import jax
import jax.numpy as jnp
from jax.experimental import pallas as pl
from jax.experimental.pallas import tpu as pltpu

PATCH = 8
VDIM = 8
OUT_DIM = 768
NSIDE = 12
NPATCH = NSIDE ** 3
VPP = PATCH ** 3
SLAB = NSIDE * NSIDE


def _pack_kernel(x_ref, z_ref):
    t = x_ref[0]
    xp = t.reshape(PATCH, NSIDE, PATCH, NSIDE, PATCH)
    xp = xp.transpose(1, 3, 0, 2, 4).reshape(SLAB, VPP)
    m = (xp >= -1000.0) & (xp < -75.0)
    a = jnp.where(m, xp, 0.0).astype(jnp.bfloat16)
    nb = jnp.where(m, 0.0, 1.0).astype(jnp.bfloat16)
    z_ref[0] = jnp.concatenate([a, nb], axis=1)


def _gemm_kernel(z_ref, g_ref, b_ref, out_ref):
    dn = (((1,), (1,)), ((), ()))
    acc = jax.lax.dot_general(g_ref[...], z_ref[0], dn,
                              preferred_element_type=jnp.float32)
    out_ref[0] = acc + b_ref[...]


def kernel(x, vectors, cls_vectors, fc_w, fc_b):
    B = x.shape[0]
    xs = x.reshape(B, 96, 96, 96)
    tv9 = jnp.tile(vectors[9], VPP)[None, :]
    def _wsum(w):
        return jax.lax.reduce_window(
            w, 0.0, jax.lax.add,
            window_dimensions=(1, VDIM), window_strides=(1, VDIM),
            padding="VALID")
    s = _wsum(fc_w)
    uv = _wsum(fc_w * tv9)
    g = jnp.concatenate([s, uv], axis=1).astype(jnp.bfloat16)
    b2 = fc_b.reshape(OUT_DIM, 1)

    z = pl.pallas_call(
        _pack_kernel,
        grid=(B, NSIDE),
        in_specs=[
            pl.BlockSpec((1, PATCH, 96, 96), lambda b, nd: (b, nd, 0, 0)),
        ],
        out_specs=pl.BlockSpec((1, SLAB, 2 * VPP), lambda b, nd: (b, nd, 0)),
        out_shape=jax.ShapeDtypeStruct((B, NPATCH, 2 * VPP), jnp.bfloat16),
        compiler_params=pltpu.CompilerParams(
            dimension_semantics=("parallel", "parallel"),
        ),
    )(xs)

    out = pl.pallas_call(
        _gemm_kernel,
        grid=(B,),
        in_specs=[
            pl.BlockSpec((1, NPATCH, 2 * VPP), lambda b: (b, 0, 0)),
            pl.BlockSpec((OUT_DIM, 2 * VPP), lambda b: (0, 0)),
            pl.BlockSpec((OUT_DIM, 1), lambda b: (0, 0)),
        ],
        out_specs=pl.BlockSpec((1, OUT_DIM, NPATCH), lambda b: (b, 0, 0)),
        out_shape=jax.ShapeDtypeStruct((B, OUT_DIM, NPATCH), jnp.float32),
        compiler_params=pltpu.CompilerParams(
            dimension_semantics=("parallel",),
        ),
    )(z, g, b2)

    return out.reshape(B, OUT_DIM, NSIDE, NSIDE, NSIDE)


# speedup vs baseline: 1.0806x; 1.0806x over previous
import jax
import jax.numpy as jnp
from jax.experimental import pallas as pl
from jax.experimental.pallas import tpu as pltpu

PATCH = 8
VDIM = 8
OUT_DIM = 768
NSIDE = 12
NPATCH = NSIDE ** 3
VPP = PATCH ** 3
SLAB = NSIDE * NSIDE


def _pack_kernel(x_ref, z_ref):
    t = x_ref[0]
    xp = t.reshape(PATCH, NSIDE, PATCH, NSIDE, PATCH)
    xp = xp.transpose(1, 3, 0, 2, 4).reshape(SLAB, VPP)
    m = (xp >= -1000.0) & (xp < -75.0)
    a = jnp.where(m, xp, 0.0).astype(jnp.bfloat16)
    nb = jnp.where(m, 0.0, 1.0).astype(jnp.bfloat16)
    z_ref[0] = jnp.concatenate([a, nb], axis=1)


def _gemm_kernel(z_ref, g_ref, b_ref, out_ref):
    dn = (((1,), (1,)), ((), ()))
    acc = jax.lax.dot_general(g_ref[...], z_ref[0], dn,
                              preferred_element_type=jnp.float32)
    out_ref[0] = acc + b_ref[...]


def kernel(x, vectors, cls_vectors, fc_w, fc_b):
    B = x.shape[0]
    xs = x.reshape(B, 96, 96, 96)
    w3 = fc_w.reshape(OUT_DIM, VPP, VDIM)
    s = w3.sum(-1)
    uv = w3 @ vectors[9]
    g = jnp.concatenate([s, uv], axis=1).astype(jnp.bfloat16)
    b2 = fc_b.reshape(OUT_DIM, 1)

    z = pl.pallas_call(
        _pack_kernel,
        grid=(B, NSIDE),
        in_specs=[
            pl.BlockSpec((1, PATCH, 96, 96), lambda b, nd: (b, nd, 0, 0)),
        ],
        out_specs=pl.BlockSpec((1, SLAB, 2 * VPP), lambda b, nd: (b, nd, 0)),
        out_shape=jax.ShapeDtypeStruct((B, NPATCH, 2 * VPP), jnp.bfloat16),
        compiler_params=pltpu.CompilerParams(
            dimension_semantics=("parallel", "parallel"),
        ),
    )(xs)

    out = pl.pallas_call(
        _gemm_kernel,
        grid=(B,),
        in_specs=[
            pl.BlockSpec((1, NPATCH, 2 * VPP), lambda b: (b, 0, 0)),
            pl.BlockSpec((OUT_DIM, 2 * VPP), lambda b: (0, 0)),
            pl.BlockSpec((OUT_DIM, 1), lambda b: (0, 0)),
        ],
        out_specs=pl.BlockSpec((1, OUT_DIM, NPATCH), lambda b: (b, 0, 0)),
        out_shape=jax.ShapeDtypeStruct((B, OUT_DIM, NPATCH), jnp.float32),
        compiler_params=pltpu.CompilerParams(
            dimension_semantics=("parallel",),
        ),
    )(z, g, b2)

    return out.reshape(B, OUT_DIM, NSIDE, NSIDE, NSIDE)
